# Initial kernel scaffold; baseline (speedup 1.0000x reference)
import jax
import jax.numpy as jnp
from jax.experimental import pallas as pl


def kernel(x, W_enc, b_enc, W_dec_q, b_dec_q, W_dec_e, b_dec_e, embeddings):
    raise NotImplementedError("write your pallas kernel here")


# speedup vs baseline: 2.5139x; 2.5139x over previous
import functools

import jax
import jax.numpy as jnp
from jax import lax
from jax.experimental import pallas as pl
from jax.experimental.pallas import tpu as pltpu
from jax.experimental.pallas import tpu_sc as plsc

B = 1024
D_IN = 512
LATENT = 64
SOM_R = 32
SOM_C = 32
NCODE = SOM_R * SOM_C
BLK = 256
GRID = B // BLK
NSLOT = 8

_BIG = (1 << 30)


def _tc_body(x_ref, we_ref, be_ref, wq_ref, bq_ref, wde_ref, bde_ref,
             e_ref, et_ref,
             xe_ref, xq_ref, ze_ref, zq_ref, zd_ref, k_ref, g_ref):
    x = x_ref[...]
    E = e_ref[...]
    Et = et_ref[...]
    ze = jnp.dot(x, we_ref[...], preferred_element_type=jnp.float32) + be_ref[...]
    ze_ref[...] = ze

    enorm = jnp.sum(Et * Et, axis=0, keepdims=True)
    cross = jnp.dot(ze, Et, preferred_element_type=jnp.float32,
                    precision=lax.Precision.HIGHEST)
    c = enorm - 2.0 * cross
    znorm = jnp.sum(ze * ze, axis=1, keepdims=True)
    zd_ref[...] = znorm + c

    cmin = jnp.min(c, axis=1, keepdims=True)
    iota = lax.broadcasted_iota(jnp.int32, (BLK, NCODE), 1)
    k = jnp.min(jnp.where(c <= cmin, iota, _BIG), axis=1, keepdims=True)
    k_ref[...] = k

    onehot = (iota == k).astype(jnp.float32)
    zq = jnp.dot(onehot, E, preferred_element_type=jnp.float32,
                 precision=lax.Precision.HIGHEST)
    zq_ref[...] = zq
    xq_ref[...] = jnp.dot(zq, wq_ref[...], preferred_element_type=jnp.float32) + bq_ref[...]
    xe_ref[...] = jnp.dot(ze, wde_ref[...], preferred_element_type=jnp.float32) + bde_ref[...]

    k1 = k // SOM_C
    k2 = k % SOM_C
    up = jnp.where(k1 < (SOM_R - 1), k + SOM_C, NCODE)
    down = jnp.where(k1 > 0, k - SOM_C, NCODE)
    left = jnp.where(k2 > 0, k - 1, NCODE)
    pad = jnp.zeros((BLK, NSLOT - 3), jnp.int32)
    g_ref[...] = jnp.concatenate([up, down, left, pad], axis=1)


def _tc_call(x, W_enc, b_enc2, W_dec_q, b_dec_q2, W_dec_e, b_dec_e2,
             E_flat, E_t):
    full = lambda s: pl.BlockSpec(s, lambda i: (0,) * len(s))
    return pl.pallas_call(
        _tc_body,
        grid=(GRID,),
        in_specs=[
            pl.BlockSpec((BLK, D_IN), lambda i: (i, 0)),
            full((D_IN, LATENT)),
            full((1, LATENT)),
            full((LATENT, D_IN)),
            full((1, D_IN)),
            full((LATENT, D_IN)),
            full((1, D_IN)),
            full((NCODE, LATENT)),
            full((LATENT, NCODE)),
        ],
        out_specs=[
            pl.BlockSpec((BLK, D_IN), lambda i: (i, 0)),
            pl.BlockSpec((BLK, D_IN), lambda i: (i, 0)),
            pl.BlockSpec((BLK, LATENT), lambda i: (i, 0)),
            pl.BlockSpec((BLK, LATENT), lambda i: (i, 0)),
            pl.BlockSpec((BLK, NCODE), lambda i: (i, 0)),
            pl.BlockSpec((BLK, 1), lambda i: (i, 0)),
            pl.BlockSpec((BLK, NSLOT), lambda i: (i, 0)),
        ],
        out_shape=[
            jax.ShapeDtypeStruct((B, D_IN), jnp.float32),
            jax.ShapeDtypeStruct((B, D_IN), jnp.float32),
            jax.ShapeDtypeStruct((B, LATENT), jnp.float32),
            jax.ShapeDtypeStruct((B, LATENT), jnp.float32),
            jax.ShapeDtypeStruct((B, NCODE), jnp.float32),
            jax.ShapeDtypeStruct((B, 1), jnp.int32),
            jax.ShapeDtypeStruct((B, NSLOT), jnp.int32),
        ],
        compiler_params=pltpu.CompilerParams(
            dimension_semantics=("arbitrary",),
        ),
    )(x, W_enc, b_enc2, W_dec_q, b_dec_q2, W_dec_e, b_dec_e2, E_flat, E_t)


_NC = 2
_NS = 16
_NW = _NC * _NS
_NG = 3 * B
_BPW = _NG // _NW

_TW = 128


@functools.lru_cache(maxsize=None)
def _sc_gather_fn():
    mesh = plsc.VectorSubcoreMesh(core_axis_name="c", subcore_axis_name="s")

    @functools.partial(
        pl.kernel,
        mesh=mesh,
        out_type=jax.ShapeDtypeStruct((_NG, _TW), jnp.float32),
        scratch_types=[
            pltpu.VMEM((_BPW,), jnp.int32),
            pltpu.VMEM((_BPW, _TW), jnp.float32),
            pltpu.SemaphoreType.DMA,
        ],
    )
    def _sc_gather(table_hbm, idx_hbm, out_hbm, idx_v, rows_v, sem):
        wid = lax.axis_index("s") * _NC + lax.axis_index("c")
        base = wid * _BPW
        pltpu.sync_copy(idx_hbm.at[pl.ds(base, _BPW)], idx_v)
        pltpu.async_copy(table_hbm.at[idx_v], rows_v, sem).wait()
        pltpu.sync_copy(rows_v, out_hbm.at[pl.ds(base, _BPW)])

    return _sc_gather


def kernel(x, W_enc, b_enc, W_dec_q, b_dec_q, W_dec_e, b_dec_e, embeddings):
    E_flat = embeddings.reshape(NCODE, LATENT)
    x_e, x_q, z_e, z_q, zdist, k2d, gidx = _tc_call(
        x, W_enc, b_enc.reshape(1, LATENT),
        W_dec_q, b_dec_q.reshape(1, D_IN),
        W_dec_e, b_dec_e.reshape(1, D_IN),
        E_flat, E_flat.T)
    k = k2d.reshape(B)

    E_pad = jnp.pad(E_flat, ((0, 8), (0, _TW - LATENT)))
    idx_flat = gidx[:, :3].T.reshape(_NG)
    nb = _sc_gather_fn()(E_pad, idx_flat)[:, :LATENT].reshape(3, B, LATENT)
    zeros = jnp.zeros((B, LATENT), jnp.float32)
    z_q_neighbors = jnp.stack([z_q, nb[0], nb[1], zeros, nb[2]], axis=1)
    return (x_e, x_q, z_e, z_q, z_q_neighbors, k, zdist)
